# baseline (speedup 1.0000x reference)
import dataclasses as _dc

import jax
import jax.numpy as jnp
from jax.experimental import pallas as pl
from jax.experimental.pallas import tpu as pltpu
from jax.experimental.pallas import tpu_sc as plsc

N_SIDE = 10000
E = 320000
IN_DIM = 128
HID = 128
NH = 4
N_TOT = 2 * N_SIDE
NPG = N_SIDE // 8

_ROWS = 200


def _align_body(q_ref, kt_ref, v_ref, o_ref):
    att = jnp.dot(q_ref[...], kt_ref[...], preferred_element_type=jnp.float32)
    m = jnp.max(att, axis=-1, keepdims=True)
    e = jnp.exp(att - m)
    s = jnp.sum(e, axis=-1, keepdims=True)
    o_ref[...] = jnp.dot(e / s, v_ref[...], preferred_element_type=jnp.float32)


@jax.jit
def _align(q, kt, v):
    return pl.pallas_call(
        _align_body,
        grid=(N_SIDE // _ROWS,),
        in_specs=[
            pl.BlockSpec((_ROWS, IN_DIM), lambda i: (i, 0)),
            pl.BlockSpec((IN_DIM, N_SIDE), lambda i: (0, 0)),
            pl.BlockSpec((N_SIDE, IN_DIM), lambda i: (0, 0)),
        ],
        out_specs=pl.BlockSpec((_ROWS, IN_DIM), lambda i: (i, 0)),
        out_shape=jax.ShapeDtypeStruct((N_SIDE, IN_DIM), jnp.float32),
    )(q, kt, v)


def _leaky(x):
    return jnp.where(x >= 0, x, 0.2 * x)


_NW = 32
_AROWS = 128
_NROUND = 5
_SBLK = 4000
_NB = E // _SBLK
_LCAP = 48
_FIRE = _LCAP - 16
_FDIM = NH * HID


def _agg_body(vals_hbm, dst_hbm, zeros_hbm, out_hbm,
              dbufA, dbufB, stagE, stagR, listeA, listrA, listeB, listrB,
              rows_vA, rows_vB, acc, nref,
              semA, semB, gsemA, gsemB):
    c = jax.lax.axis_index("c")
    s = jax.lax.axis_index("s")
    wid = c * 16 + s
    iota16 = jax.lax.iota(jnp.int32, 16)
    zero16 = jnp.zeros((16,), jnp.int32)

    def clear_stag():
        for q in range(_LCAP // 16):
            stagE[pl.ds(16 * q, 16)] = zero16

    def drain(liste, listr, rows_v, gsem):
        pltpu.make_async_copy(vals_hbm.at[liste], rows_v, gsem).wait()
        nn = nref[3]

        def acc_edge(i, carry):
            r = listr[pl.ds(i, 16)][0]
            for v in range(_FDIM // 16):
                plsc.addupdate(acc.at[r, pl.ds(16 * v, 16)],
                               rows_v[i, pl.ds(16 * v, 16)])
            return carry

        jax.lax.fori_loop(0, nn, acc_edge, jnp.int32(0))
        nref[2] = jnp.int32(0)

    def launch(liste, listr, rows_v, gsem):
        for q in range(_LCAP // 16):
            liste[pl.ds(16 * q, 16)] = stagE[pl.ds(16 * q, 16)]
            listr[pl.ds(16 * q, 16)] = stagR[pl.ds(16 * q, 16)]
        pltpu.async_copy(vals_hbm.at[liste], rows_v, gsem)

    def fire():
        @pl.when(nref[1] == 0)
        def _():
            launch(listeA, listrA, rows_vA, gsemA)

            @pl.when(nref[2] == 1)
            def _():
                drain(listeB, listrB, rows_vB, gsemB)

        @pl.when(nref[1] == 1)
        def _():
            launch(listeB, listrB, rows_vB, gsemB)

            @pl.when(nref[2] == 1)
            def _():
                drain(listeA, listrA, rows_vA, gsemA)

        nref[3] = nref[0]
        nref[2] = jnp.int32(1)
        nref[1] = 1 - nref[1]
        nref[0] = jnp.int32(0)
        clear_stag()

    def flush():
        @pl.when((nref[2] == 1) & (nref[1] == 1))
        def _():
            drain(listeA, listrA, rows_vA, gsemA)

        @pl.when((nref[2] == 1) & (nref[1] == 0))
        def _():
            drain(listeB, listrB, rows_vB, gsemB)

        @pl.when(nref[0] > 0)
        def _():
            nref[3] = nref[0]

            @pl.when(nref[1] == 0)
            def _():
                launch(listeA, listrA, rows_vA, gsemA)
                drain(listeA, listrA, rows_vA, gsemA)

            @pl.when(nref[1] == 1)
            def _():
                launch(listeB, listrB, rows_vB, gsemB)
                drain(listeB, listrB, rows_vB, gsemB)
            nref[0] = jnp.int32(0)
            clear_stag()

    def round_body(R, round_carry):
        lo = (R * _NW + wid) * _AROWS
        hi = jnp.minimum(lo + _AROWS, N_TOT)
        pltpu.sync_copy(zeros_hbm, acc)
        clear_stag()
        nref[0] = jnp.int32(0)
        nref[1] = jnp.int32(0)
        nref[2] = jnp.int32(0)
        nref[3] = jnp.int32(0)

        def scan(j, buf, carry):
            base = j * _SBLK

            def vec(i, carry2):
                off = i * 16
                d16 = buf[pl.ds(off, 16)]
                inb = (d16 >= lo) & (d16 < hi)
                n0 = nref[0]
                plsc.store_compressed(stagE.at[pl.ds(n0, 16)],
                                      base + off + iota16, mask=inb)
                plsc.store_compressed(stagR.at[pl.ds(n0, 16)],
                                      d16 - lo, mask=inb)
                n1 = n0 + jnp.sum(inb.astype(jnp.int32))
                nref[0] = n1

                @pl.when(n1 > _FIRE)
                def _():
                    fire()
                return carry2

            return jax.lax.fori_loop(0, _SBLK // 16, vec, carry)

        pltpu.async_copy(dst_hbm.at[pl.ds(0, _SBLK)], dbufA, semA)

        def pair(p, carry):
            j0 = 2 * p
            pltpu.make_async_copy(
                dst_hbm.at[pl.ds(j0 * _SBLK, _SBLK)], dbufA, semA).wait()
            pltpu.async_copy(
                dst_hbm.at[pl.ds((j0 + 1) * _SBLK, _SBLK)], dbufB, semB)
            carry = scan(j0, dbufA, carry)

            @pl.when(p < _NB // 2 - 1)
            def _():
                pltpu.async_copy(
                    dst_hbm.at[pl.ds((j0 + 2) * _SBLK, _SBLK)], dbufA, semA)
            pltpu.make_async_copy(
                dst_hbm.at[pl.ds((j0 + 1) * _SBLK, _SBLK)], dbufB, semB).wait()
            carry = scan(j0 + 1, dbufB, carry)
            return carry

        jax.lax.fori_loop(0, _NB // 2, pair, jnp.int32(0))
        flush()

        @pl.when(lo + _AROWS <= N_TOT)
        def _():
            pltpu.sync_copy(acc, out_hbm.at[pl.ds(lo, _AROWS)])

        @pl.when((lo < N_TOT) & (lo + _AROWS > N_TOT))
        def _():
            pltpu.sync_copy(acc.at[pl.ds(0, N_TOT % _AROWS)],
                            out_hbm.at[pl.ds(lo, N_TOT % _AROWS)])
        return round_carry

    jax.lax.fori_loop(0, _NROUND, round_body, jnp.int32(0))


_sc_params = pltpu.CompilerParams()
if "needs_layout_passes" in pltpu.CompilerParams.__dataclass_fields__:
    _sc_params = _dc.replace(_sc_params, needs_layout_passes=False)

_scatter_agg = pl.kernel(
    _agg_body,
    out_type=jax.ShapeDtypeStruct((N_TOT, _FDIM), jnp.float32),
    mesh=plsc.VectorSubcoreMesh(core_axis_name="c", subcore_axis_name="s"),
    compiler_params=_sc_params,
    scratch_types=[
        pltpu.VMEM((_SBLK,), jnp.int32),
        pltpu.VMEM((_SBLK,), jnp.int32),
        pltpu.VMEM((_LCAP,), jnp.int32),
        pltpu.VMEM((_LCAP,), jnp.int32),
        pltpu.VMEM((_LCAP,), jnp.int32),
        pltpu.VMEM((_LCAP + 16,), jnp.int32),
        pltpu.VMEM((_LCAP,), jnp.int32),
        pltpu.VMEM((_LCAP + 16,), jnp.int32),
        pltpu.VMEM((_LCAP, _FDIM), jnp.float32),
        pltpu.VMEM((_LCAP, _FDIM), jnp.float32),
        pltpu.VMEM((_AROWS, _FDIM), jnp.float32),
        pltpu.SMEM((4,), jnp.int32),
        pltpu.SemaphoreType.DMA,
        pltpu.SemaphoreType.DMA,
        pltpu.SemaphoreType.DMA,
        pltpu.SemaphoreType.DMA,
    ],
)


_SROWS = 640
_STAIL = N_TOT - 31 * _SROWS
_L2 = 512
_F2 = _L2 - 16


def _stats_body(epad_hbm, dst_hbm, minit_hbm, zinit_hbm, m_hbm, s_hbm,
                dbufA, dbufB, stagE, stagR, stagS, rows_e, mloc, sloc, nref,
                semA, semB, gsem):
    c = jax.lax.axis_index("c")
    s = jax.lax.axis_index("s")
    wid = c * 16 + s
    lo = wid * _SROWS
    hi = jnp.minimum(lo + _SROWS, N_TOT)
    iota16 = jax.lax.iota(jnp.int32, 16)
    zero16 = jnp.zeros((16,), jnp.int32)

    def clear_stag():
        for q in range(_L2 // 16):
            stagE[pl.ds(16 * q, 16)] = zero16

    def scan_pass(update):
        def fire():
            pltpu.async_copy(epad_hbm.at[stagE], rows_e, gsem).wait()
            nn = nref[0]

            def per_edge(i, carry):
                r = stagR[pl.ds(i, 16)][0]
                sub = stagS[pl.ds(i, 16)][0]
                ivec = jnp.full((16,), i, jnp.int32)
                erow = plsc.load_gather(rows_e, [ivec, sub * 16 + iota16])
                rr = jnp.full((16,), r >> 3, jnp.int32)
                cc = (r & 7) * 16 + iota16
                update(rr, cc, erow)
                return carry

            jax.lax.fori_loop(0, nn, per_edge, jnp.int32(0))
            clear_stag()
            nref[0] = jnp.int32(0)

        def scan(j, buf, carry):
            base = j * _SBLK

            def vec(i, carry2):
                off = i * 16
                d16 = buf[pl.ds(off, 16)]
                inb = (d16 >= lo) & (d16 < hi)
                n0 = nref[0]
                eidx = base + off + iota16
                plsc.store_compressed(stagE.at[pl.ds(n0, 16)],
                                      eidx >> 3, mask=inb)
                plsc.store_compressed(stagS.at[pl.ds(n0, 16)],
                                      eidx & 7, mask=inb)
                plsc.store_compressed(stagR.at[pl.ds(n0, 16)],
                                      d16 - lo, mask=inb)
                nref[0] = n0 + jnp.sum(inb.astype(jnp.int32))

                @pl.when(nref[0] > _F2)
                def _():
                    fire()
                return carry2

            return jax.lax.fori_loop(0, _SBLK // 16, vec, carry)

        clear_stag()
        nref[0] = jnp.int32(0)
        pltpu.async_copy(dst_hbm.at[pl.ds(0, _SBLK)], dbufA, semA)

        def pair(p, carry):
            j0 = 2 * p
            pltpu.make_async_copy(
                dst_hbm.at[pl.ds(j0 * _SBLK, _SBLK)], dbufA, semA).wait()
            pltpu.async_copy(
                dst_hbm.at[pl.ds((j0 + 1) * _SBLK, _SBLK)], dbufB, semB)
            carry = scan(j0, dbufA, carry)

            @pl.when(p < _NB // 2 - 1)
            def _():
                pltpu.async_copy(
                    dst_hbm.at[pl.ds((j0 + 2) * _SBLK, _SBLK)], dbufA, semA)
            pltpu.make_async_copy(
                dst_hbm.at[pl.ds((j0 + 1) * _SBLK, _SBLK)], dbufB, semB).wait()
            carry = scan(j0 + 1, dbufB, carry)
            return carry

        jax.lax.fori_loop(0, _NB // 2, pair, jnp.int32(0))

        @pl.when(nref[0] > 0)
        def _():
            fire()

    pltpu.sync_copy(minit_hbm, mloc)
    pltpu.sync_copy(zinit_hbm, sloc)

    def upd_max(rr, cc, erow):
        old = plsc.load_gather(mloc, [rr, cc])
        plsc.store_scatter(mloc, [rr, cc], jnp.maximum(old, erow))

    scan_pass(upd_max)

    def upd_sum(rr, cc, erow):
        mrow = plsc.load_gather(mloc, [rr, cc])
        old = plsc.load_gather(sloc, [rr, cc])
        plsc.store_scatter(sloc, [rr, cc], old + jnp.exp(erow - mrow))

    scan_pass(upd_sum)

    @pl.when(lo + _SROWS <= N_TOT)
    def _():
        pltpu.sync_copy(mloc, m_hbm.at[pl.ds(wid * (_SROWS // 8), _SROWS // 8)])
        pltpu.sync_copy(sloc, s_hbm.at[pl.ds(wid * (_SROWS // 8), _SROWS // 8)])

    @pl.when(lo + _SROWS > N_TOT)
    def _():
        pltpu.sync_copy(mloc.at[pl.ds(0, 24)],
                        m_hbm.at[pl.ds(wid * (_SROWS // 8), 24)])
        pltpu.sync_copy(sloc.at[pl.ds(0, 24)],
                        s_hbm.at[pl.ds(wid * (_SROWS // 8), 24)])


_edge_stats = pl.kernel(
    _stats_body,
    out_type=(jax.ShapeDtypeStruct((N_TOT // 8 + 4, 128), jnp.float32),
              jax.ShapeDtypeStruct((N_TOT // 8 + 4, 128), jnp.float32)),
    mesh=plsc.VectorSubcoreMesh(core_axis_name="c", subcore_axis_name="s"),
    compiler_params=_sc_params,
    scratch_types=[
        pltpu.VMEM((_SBLK,), jnp.int32),
        pltpu.VMEM((_SBLK,), jnp.int32),
        pltpu.VMEM((_L2,), jnp.int32),
        pltpu.VMEM((_L2 + 16,), jnp.int32),
        pltpu.VMEM((_L2 + 16,), jnp.int32),
        pltpu.VMEM((_L2, 128), jnp.float32),
        pltpu.VMEM((_SROWS // 8, 128), jnp.float32),
        pltpu.VMEM((_SROWS // 8, 128), jnp.float32),
        pltpu.SMEM((1,), jnp.int32),
        pltpu.SemaphoreType.DMA,
        pltpu.SemaphoreType.DMA,
        pltpu.SemaphoreType.DMA,
    ],
)


def _gat(h, src, dst, W, al, ar, zeros, minit, zinit):
    N = h.shape[0]
    ft = (h @ W).reshape(N, NH, HID)
    a1 = jnp.einsum('nhd,hdo->nho', ft, al)
    a2 = jnp.einsum('nhd,hdo->nho', ft, ar)
    e = _leaky(a1[src] + a2[dst])
    epad = jnp.concatenate(
        [e.reshape(E, NH), jnp.zeros((E, 16 - NH), jnp.float32)],
        axis=1).reshape(E // 8, 128)
    m_tab, s_tab = _edge_stats(epad, dst, minit, zinit)
    emax = m_tab.reshape(-1, 16)[:N, :NH].reshape(N, NH, 1)
    ee = jnp.exp(e - emax[dst])
    esum = s_tab.reshape(-1, 16)[:N, :NH].reshape(N, NH, 1)
    a = ee / (esum[dst] + 1e-9)
    vals = (ft[src] * a).reshape(E, _FDIM)
    out = _scatter_agg(vals, dst, zeros)
    return jax.nn.elu(out)


def _bn_eval(x, g, b):
    return x / jnp.sqrt(1.0 + 1e-5) * g + b


def kernel(x1, x2, edge_index, edge_embeddings, W1, attn_l1, attn_r1, W2,
           attn_l2, attn_r2, bn1_g, bn1_b, lin1_W, lin1_b, bn2_g, bn2_b,
           lin2_W, lin2_b):
    src = edge_index[0].astype(jnp.int32)
    dst = edge_index[1].astype(jnp.int32)
    zeros = jnp.zeros((_AROWS, _FDIM), jnp.float32)
    minit = jnp.full((_SROWS // 8, 128), -3.0e38, jnp.float32)
    zinit = jnp.zeros((_SROWS // 8, 128), jnp.float32)
    x1a = _align(x1, x2.T, x2)
    x2a = _align(x2, x1.T, x1)
    q1 = jnp.concatenate([x1, x1a, x1 - x1a, x1 * x1a], axis=-1)
    q2 = jnp.concatenate([x2, x2a, x2 - x2a, x2 * x2a], axis=-1)
    h = jnp.concatenate([q1, q2], axis=0)
    h = _gat(h, src, dst, W1, attn_l1, attn_r1, zeros, minit, zinit)
    h = _gat(h, src, dst, W2, attn_l2, attn_r2, zeros, minit, zinit)
    hcat = jnp.concatenate([h, edge_embeddings], axis=1)
    gid = jnp.arange(N_TOT) // NPG
    hg = jax.ops.segment_sum(hcat, gid, num_segments=16) / float(NPG)
    xcls = jnp.concatenate([hg[:8], hg[8:]], axis=1)
    y = _bn_eval(xcls, bn1_g, bn1_b)
    y = y @ lin1_W + lin1_b
    y = jax.nn.relu(y)
    y = _bn_eval(y, bn2_g, bn2_b)
    return y @ lin2_W + lin2_b


# speedup vs baseline: 1.0044x; 1.0044x over previous
import dataclasses as _dc

import jax
import jax.numpy as jnp
from jax.experimental import pallas as pl
from jax.experimental.pallas import tpu as pltpu
from jax.experimental.pallas import tpu_sc as plsc

N_SIDE = 10000
E = 320000
IN_DIM = 128
HID = 128
NH = 4
N_TOT = 2 * N_SIDE
NPG = N_SIDE // 8

_ROWS = 200


def _align_body(q_ref, kt_ref, v_ref, o_ref):
    att = jnp.dot(q_ref[...], kt_ref[...], preferred_element_type=jnp.float32)
    m = jnp.max(att, axis=-1, keepdims=True)
    e = jnp.exp(att - m)
    s = jnp.sum(e, axis=-1, keepdims=True)
    o_ref[...] = jnp.dot(e / s, v_ref[...], preferred_element_type=jnp.float32)


@jax.jit
def _align(q, kt, v):
    return pl.pallas_call(
        _align_body,
        grid=(N_SIDE // _ROWS,),
        in_specs=[
            pl.BlockSpec((_ROWS, IN_DIM), lambda i: (i, 0)),
            pl.BlockSpec((IN_DIM, N_SIDE), lambda i: (0, 0)),
            pl.BlockSpec((N_SIDE, IN_DIM), lambda i: (0, 0)),
        ],
        out_specs=pl.BlockSpec((_ROWS, IN_DIM), lambda i: (i, 0)),
        out_shape=jax.ShapeDtypeStruct((N_SIDE, IN_DIM), jnp.float32),
    )(q, kt, v)


def _leaky(x):
    return jnp.where(x >= 0, x, 0.2 * x)


_NW = 32
_AROWS = 128
_NROUND = 5
_SBLK = 4000
_NB = E // _SBLK
_LCAP = 48
_FIRE = _LCAP - 16
_FDIM = NH * HID


def _agg_body(vals_hbm, dst_hbm, zeros_hbm, out_hbm,
              dbufA, dbufB, stagE, stagR, listeA, listrA, listeB, listrB,
              rows_vA, rows_vB, acc, nref,
              semA, semB, gsemA, gsemB):
    c = jax.lax.axis_index("c")
    s = jax.lax.axis_index("s")
    wid = c * 16 + s
    iota16 = jax.lax.iota(jnp.int32, 16)
    zero16 = jnp.zeros((16,), jnp.int32)

    def clear_stag():
        for q in range(_LCAP // 16):
            stagE[pl.ds(16 * q, 16)] = zero16

    def drain(liste, listr, rows_v, gsem):
        pltpu.make_async_copy(vals_hbm.at[liste], rows_v, gsem).wait()
        nn = nref[3]

        def acc_edge(i, carry):
            r = listr[pl.ds(i, 16)][0]
            for v in range(_FDIM // 16):
                plsc.addupdate(acc.at[r, pl.ds(16 * v, 16)],
                               rows_v[i, pl.ds(16 * v, 16)])
            return carry

        jax.lax.fori_loop(0, nn, acc_edge, jnp.int32(0))
        nref[2] = jnp.int32(0)

    def launch(liste, listr, rows_v, gsem):
        for q in range(_LCAP // 16):
            liste[pl.ds(16 * q, 16)] = stagE[pl.ds(16 * q, 16)]
            listr[pl.ds(16 * q, 16)] = stagR[pl.ds(16 * q, 16)]
        pltpu.async_copy(vals_hbm.at[liste], rows_v, gsem)

    def fire():
        @pl.when(nref[1] == 0)
        def _():
            launch(listeA, listrA, rows_vA, gsemA)

            @pl.when(nref[2] == 1)
            def _():
                drain(listeB, listrB, rows_vB, gsemB)

        @pl.when(nref[1] == 1)
        def _():
            launch(listeB, listrB, rows_vB, gsemB)

            @pl.when(nref[2] == 1)
            def _():
                drain(listeA, listrA, rows_vA, gsemA)

        nref[3] = nref[0]
        nref[2] = jnp.int32(1)
        nref[1] = 1 - nref[1]
        nref[0] = jnp.int32(0)
        clear_stag()

    def flush():
        @pl.when((nref[2] == 1) & (nref[1] == 1))
        def _():
            drain(listeA, listrA, rows_vA, gsemA)

        @pl.when((nref[2] == 1) & (nref[1] == 0))
        def _():
            drain(listeB, listrB, rows_vB, gsemB)

        @pl.when(nref[0] > 0)
        def _():
            nref[3] = nref[0]

            @pl.when(nref[1] == 0)
            def _():
                launch(listeA, listrA, rows_vA, gsemA)
                drain(listeA, listrA, rows_vA, gsemA)

            @pl.when(nref[1] == 1)
            def _():
                launch(listeB, listrB, rows_vB, gsemB)
                drain(listeB, listrB, rows_vB, gsemB)
            nref[0] = jnp.int32(0)
            clear_stag()

    def round_body(R, round_carry):
        lo = (R * _NW + wid) * _AROWS
        hi = jnp.minimum(lo + _AROWS, N_TOT)
        pltpu.sync_copy(zeros_hbm, acc)
        clear_stag()
        nref[0] = jnp.int32(0)
        nref[1] = jnp.int32(0)
        nref[2] = jnp.int32(0)
        nref[3] = jnp.int32(0)

        def scan(j, buf, carry):
            base = j * _SBLK

            def vec(i, carry2):
                off = i * 16
                d16 = buf[pl.ds(off, 16)]
                inb = (d16 >= lo) & (d16 < hi)
                n0 = nref[0]
                plsc.store_compressed(stagE.at[pl.ds(n0, 16)],
                                      base + off + iota16, mask=inb)
                plsc.store_compressed(stagR.at[pl.ds(n0, 16)],
                                      d16 - lo, mask=inb)
                n1 = n0 + jnp.sum(inb.astype(jnp.int32))
                nref[0] = n1

                @pl.when(n1 > _FIRE)
                def _():
                    fire()
                return carry2

            return jax.lax.fori_loop(0, _SBLK // 16, vec, carry)

        pltpu.async_copy(dst_hbm.at[pl.ds(0, _SBLK)], dbufA, semA)

        def pair(p, carry):
            j0 = 2 * p
            pltpu.make_async_copy(
                dst_hbm.at[pl.ds(j0 * _SBLK, _SBLK)], dbufA, semA).wait()
            pltpu.async_copy(
                dst_hbm.at[pl.ds((j0 + 1) * _SBLK, _SBLK)], dbufB, semB)
            carry = scan(j0, dbufA, carry)

            @pl.when(p < _NB // 2 - 1)
            def _():
                pltpu.async_copy(
                    dst_hbm.at[pl.ds((j0 + 2) * _SBLK, _SBLK)], dbufA, semA)
            pltpu.make_async_copy(
                dst_hbm.at[pl.ds((j0 + 1) * _SBLK, _SBLK)], dbufB, semB).wait()
            carry = scan(j0 + 1, dbufB, carry)
            return carry

        jax.lax.fori_loop(0, _NB // 2, pair, jnp.int32(0))
        flush()

        @pl.when(lo + _AROWS <= N_TOT)
        def _():
            pltpu.sync_copy(acc, out_hbm.at[pl.ds(lo, _AROWS)])

        @pl.when((lo < N_TOT) & (lo + _AROWS > N_TOT))
        def _():
            pltpu.sync_copy(acc.at[pl.ds(0, N_TOT % _AROWS)],
                            out_hbm.at[pl.ds(lo, N_TOT % _AROWS)])
        return round_carry

    jax.lax.fori_loop(0, _NROUND, round_body, jnp.int32(0))


_sc_params = pltpu.CompilerParams()
if "needs_layout_passes" in pltpu.CompilerParams.__dataclass_fields__:
    _sc_params = _dc.replace(_sc_params, needs_layout_passes=False)

_scatter_agg = pl.kernel(
    _agg_body,
    out_type=jax.ShapeDtypeStruct((N_TOT, _FDIM), jnp.float32),
    mesh=plsc.VectorSubcoreMesh(core_axis_name="c", subcore_axis_name="s"),
    compiler_params=_sc_params,
    scratch_types=[
        pltpu.VMEM((_SBLK,), jnp.int32),
        pltpu.VMEM((_SBLK,), jnp.int32),
        pltpu.VMEM((_LCAP,), jnp.int32),
        pltpu.VMEM((_LCAP,), jnp.int32),
        pltpu.VMEM((_LCAP,), jnp.int32),
        pltpu.VMEM((_LCAP + 16,), jnp.int32),
        pltpu.VMEM((_LCAP,), jnp.int32),
        pltpu.VMEM((_LCAP + 16,), jnp.int32),
        pltpu.VMEM((_LCAP, _FDIM), jnp.float32),
        pltpu.VMEM((_LCAP, _FDIM), jnp.float32),
        pltpu.VMEM((_AROWS, _FDIM), jnp.float32),
        pltpu.SMEM((4,), jnp.int32),
        pltpu.SemaphoreType.DMA,
        pltpu.SemaphoreType.DMA,
        pltpu.SemaphoreType.DMA,
        pltpu.SemaphoreType.DMA,
    ],
)


_SROWS = 640
_STAIL = N_TOT - 31 * _SROWS
_L2 = 512
_F2 = _L2 - 16


def _stats_body(epad_hbm, dst_hbm, minit_hbm, zinit_hbm, m_hbm, s_hbm,
                dbufA, dbufB, stagE, stagR, stagS, rows_e, mloc, sloc, nref,
                semA, semB, gsem):
    c = jax.lax.axis_index("c")
    s = jax.lax.axis_index("s")
    wid = c * 16 + s
    lo = wid * _SROWS
    hi = jnp.minimum(lo + _SROWS, N_TOT)
    iota16 = jax.lax.iota(jnp.int32, 16)
    zero16 = jnp.zeros((16,), jnp.int32)

    def clear_stag():
        for q in range(_L2 // 16):
            stagE[pl.ds(16 * q, 16)] = zero16

    def scan_pass(update):
        def fire():
            pltpu.async_copy(epad_hbm.at[stagE], rows_e, gsem).wait()
            nn = nref[0]

            def per_edge(i, carry):
                r = stagR[pl.ds(i, 16)][0]
                sub = stagS[pl.ds(i, 16)][0]
                ivec = jnp.full((16,), i, jnp.int32)
                erow = plsc.load_gather(rows_e, [ivec, sub * 16 + iota16])
                rr = jnp.full((16,), r >> 3, jnp.int32)
                cc = (r & 7) * 16 + iota16
                update(rr, cc, erow)
                return carry

            jax.lax.fori_loop(0, nn, per_edge, jnp.int32(0))
            clear_stag()
            nref[0] = jnp.int32(0)

        def scan(j, buf, carry):
            base = j * _SBLK

            def vec(i, carry2):
                off = i * 16
                d16 = buf[pl.ds(off, 16)]
                inb = (d16 >= lo) & (d16 < hi)
                n0 = nref[0]
                eidx = base + off + iota16
                plsc.store_compressed(stagE.at[pl.ds(n0, 16)],
                                      eidx >> 3, mask=inb)
                plsc.store_compressed(stagS.at[pl.ds(n0, 16)],
                                      eidx & 7, mask=inb)
                plsc.store_compressed(stagR.at[pl.ds(n0, 16)],
                                      d16 - lo, mask=inb)
                nref[0] = n0 + jnp.sum(inb.astype(jnp.int32))

                @pl.when(nref[0] > _F2)
                def _():
                    fire()
                return carry2

            return jax.lax.fori_loop(0, _SBLK // 16, vec, carry)

        clear_stag()
        nref[0] = jnp.int32(0)
        pltpu.async_copy(dst_hbm.at[pl.ds(0, _SBLK)], dbufA, semA)

        def pair(p, carry):
            j0 = 2 * p
            pltpu.make_async_copy(
                dst_hbm.at[pl.ds(j0 * _SBLK, _SBLK)], dbufA, semA).wait()
            pltpu.async_copy(
                dst_hbm.at[pl.ds((j0 + 1) * _SBLK, _SBLK)], dbufB, semB)
            carry = scan(j0, dbufA, carry)

            @pl.when(p < _NB // 2 - 1)
            def _():
                pltpu.async_copy(
                    dst_hbm.at[pl.ds((j0 + 2) * _SBLK, _SBLK)], dbufA, semA)
            pltpu.make_async_copy(
                dst_hbm.at[pl.ds((j0 + 1) * _SBLK, _SBLK)], dbufB, semB).wait()
            carry = scan(j0 + 1, dbufB, carry)
            return carry

        jax.lax.fori_loop(0, _NB // 2, pair, jnp.int32(0))

        @pl.when(nref[0] > 0)
        def _():
            fire()

    pltpu.sync_copy(minit_hbm, mloc)
    pltpu.sync_copy(zinit_hbm, sloc)

    def upd_max(rr, cc, erow):
        old = plsc.load_gather(mloc, [rr, cc])
        plsc.store_scatter(mloc, [rr, cc], jnp.maximum(old, erow))

    scan_pass(upd_max)

    def upd_sum(rr, cc, erow):
        mrow = plsc.load_gather(mloc, [rr, cc])
        old = plsc.load_gather(sloc, [rr, cc])
        plsc.store_scatter(sloc, [rr, cc], old + jnp.exp(erow - mrow))

    scan_pass(upd_sum)

    @pl.when(lo + _SROWS <= N_TOT)
    def _():
        pltpu.sync_copy(mloc, m_hbm.at[pl.ds(wid * (_SROWS // 8), _SROWS // 8)])
        pltpu.sync_copy(sloc, s_hbm.at[pl.ds(wid * (_SROWS // 8), _SROWS // 8)])

    @pl.when(lo + _SROWS > N_TOT)
    def _():
        pltpu.sync_copy(mloc.at[pl.ds(0, 24)],
                        m_hbm.at[pl.ds(wid * (_SROWS // 8), 24)])
        pltpu.sync_copy(sloc.at[pl.ds(0, 24)],
                        s_hbm.at[pl.ds(wid * (_SROWS // 8), 24)])


_edge_stats = pl.kernel(
    _stats_body,
    out_type=(jax.ShapeDtypeStruct((N_TOT // 8 + 4, 128), jnp.float32),
              jax.ShapeDtypeStruct((N_TOT // 8 + 4, 128), jnp.float32)),
    mesh=plsc.VectorSubcoreMesh(core_axis_name="c", subcore_axis_name="s"),
    compiler_params=_sc_params,
    scratch_types=[
        pltpu.VMEM((_SBLK,), jnp.int32),
        pltpu.VMEM((_SBLK,), jnp.int32),
        pltpu.VMEM((_L2,), jnp.int32),
        pltpu.VMEM((_L2 + 16,), jnp.int32),
        pltpu.VMEM((_L2 + 16,), jnp.int32),
        pltpu.VMEM((_L2, 128), jnp.float32),
        pltpu.VMEM((_SROWS // 8, 128), jnp.float32),
        pltpu.VMEM((_SROWS // 8, 128), jnp.float32),
        pltpu.SMEM((1,), jnp.int32),
        pltpu.SemaphoreType.DMA,
        pltpu.SemaphoreType.DMA,
        pltpu.SemaphoreType.DMA,
    ],
)


def _gat(h, src, dst, W, al, ar, zeros, minit, zinit):
    N = h.shape[0]
    ft = (h @ W).reshape(N, NH, HID)
    a1 = jnp.einsum('nhd,hdo->nho', ft, al)
    a2 = jnp.einsum('nhd,hdo->nho', ft, ar)
    e = _leaky(a1[src] + a2[dst])
    emax = jax.ops.segment_max(e, dst, num_segments=N)
    emax = jnp.where(jnp.isfinite(emax), emax, 0.0)
    ee = jnp.exp(e - emax[dst])
    esum = jax.ops.segment_sum(ee, dst, num_segments=N)
    a = ee / (esum[dst] + 1e-9)
    vals = (ft[src] * a).reshape(E, _FDIM)
    out = _scatter_agg(vals, dst, zeros)
    return jax.nn.elu(out)


def _bn_eval(x, g, b):
    return x / jnp.sqrt(1.0 + 1e-5) * g + b


def kernel(x1, x2, edge_index, edge_embeddings, W1, attn_l1, attn_r1, W2,
           attn_l2, attn_r2, bn1_g, bn1_b, lin1_W, lin1_b, bn2_g, bn2_b,
           lin2_W, lin2_b):
    src = edge_index[0].astype(jnp.int32)
    dst = edge_index[1].astype(jnp.int32)
    zeros = jnp.zeros((_AROWS, _FDIM), jnp.float32)
    minit = jnp.full((_SROWS // 8, 128), -3.0e38, jnp.float32)
    zinit = jnp.zeros((_SROWS // 8, 128), jnp.float32)
    x1a = _align(x1, x2.T, x2)
    x2a = _align(x2, x1.T, x1)
    q1 = jnp.concatenate([x1, x1a, x1 - x1a, x1 * x1a], axis=-1)
    q2 = jnp.concatenate([x2, x2a, x2 - x2a, x2 * x2a], axis=-1)
    h = jnp.concatenate([q1, q2], axis=0)
    h = _gat(h, src, dst, W1, attn_l1, attn_r1, zeros, minit, zinit)
    h = _gat(h, src, dst, W2, attn_l2, attn_r2, zeros, minit, zinit)
    hcat = jnp.concatenate([h, edge_embeddings], axis=1)
    gid = jnp.arange(N_TOT) // NPG
    hg = jax.ops.segment_sum(hcat, gid, num_segments=16) / float(NPG)
    xcls = jnp.concatenate([hg[:8], hg[8:]], axis=1)
    y = _bn_eval(xcls, bn1_g, bn1_b)
    y = y @ lin1_W + lin1_b
    y = jax.nn.relu(y)
    y = _bn_eval(y, bn2_g, bn2_b)
    return y @ lin2_W + lin2_b


# speedup vs baseline: 1.1444x; 1.1395x over previous
import dataclasses as _dc

import jax
import jax.numpy as jnp
from jax.experimental import pallas as pl
from jax.experimental.pallas import tpu as pltpu
from jax.experimental.pallas import tpu_sc as plsc

N_SIDE = 10000
E = 320000
IN_DIM = 128
HID = 128
NH = 4
N_TOT = 2 * N_SIDE
NPG = N_SIDE // 8

_ROWS = 200


def _align_body(q_ref, kt_ref, v_ref, o_ref):
    att = jnp.dot(q_ref[...], kt_ref[...], preferred_element_type=jnp.float32)
    m = jnp.max(att, axis=-1, keepdims=True)
    e = jnp.exp(att - m)
    s = jnp.sum(e, axis=-1, keepdims=True)
    o_ref[...] = jnp.dot(e / s, v_ref[...], preferred_element_type=jnp.float32)


@jax.jit
def _align(q, kt, v):
    return pl.pallas_call(
        _align_body,
        grid=(N_SIDE // _ROWS,),
        in_specs=[
            pl.BlockSpec((_ROWS, IN_DIM), lambda i: (i, 0)),
            pl.BlockSpec((IN_DIM, N_SIDE), lambda i: (0, 0)),
            pl.BlockSpec((N_SIDE, IN_DIM), lambda i: (0, 0)),
        ],
        out_specs=pl.BlockSpec((_ROWS, IN_DIM), lambda i: (i, 0)),
        out_shape=jax.ShapeDtypeStruct((N_SIDE, IN_DIM), jnp.float32),
    )(q, kt, v)


def _leaky(x):
    return jnp.where(x >= 0, x, 0.2 * x)


_NW = 32
_AROWS = 128
_NROUND = 5
_SBLK = 2000
_NB = E // _SBLK
_LCAP = 112
_FIRE = _LCAP - 16
_FDIM = NH * HID


def _agg_body(vals_hbm, dst_hbm, zeros_hbm, out_hbm,
              dbufA, dbufB, liste, listr, rows_v, acc, nref,
              semA, semB, gsem):
    c = jax.lax.axis_index("c")
    s = jax.lax.axis_index("s")
    wid = c * 16 + s
    iota16 = jax.lax.iota(jnp.int32, 16)
    zero16 = jnp.zeros((16,), jnp.int32)

    def clear_list():
        for q in range(_LCAP // 16):
            liste[pl.ds(16 * q, 16)] = zero16

    def fire():
        nn = nref[0]
        pltpu.async_copy(vals_hbm.at[liste], rows_v, gsem).wait()

        def acc_edge(i, carry):
            r = listr[pl.ds(i, 16)][0]
            for v in range(_FDIM // 16):
                plsc.addupdate(acc.at[r, pl.ds(16 * v, 16)],
                               rows_v[i, pl.ds(16 * v, 16)])
            return carry

        jax.lax.fori_loop(0, nn, acc_edge, jnp.int32(0))
        clear_list()
        nref[0] = jnp.int32(0)

    for R in range(_NROUND):
        lo = (R * _NW + wid) * _AROWS
        hi = jnp.minimum(lo + _AROWS, N_TOT)
        pltpu.sync_copy(zeros_hbm, acc)
        clear_list()
        nref[0] = jnp.int32(0)

        def scan(j, buf, carry):
            base = j * _SBLK

            def vec(i, carry2):
                off = i * 16
                d16 = buf[pl.ds(off, 16)]
                inb = (d16 >= lo) & (d16 < hi)
                n0 = nref[0]
                plsc.store_compressed(liste.at[pl.ds(n0, 16)],
                                      base + off + iota16, mask=inb)
                plsc.store_compressed(listr.at[pl.ds(n0, 16)],
                                      d16 - lo, mask=inb)
                n1 = n0 + jnp.sum(inb.astype(jnp.int32))
                nref[0] = n1

                @pl.when(n1 > _FIRE)
                def _():
                    fire()
                return carry2

            return jax.lax.fori_loop(0, _SBLK // 16, vec, carry)

        pltpu.async_copy(dst_hbm.at[pl.ds(0, _SBLK)], dbufA, semA)

        def pair(p, carry):
            j0 = 2 * p
            pltpu.make_async_copy(
                dst_hbm.at[pl.ds(j0 * _SBLK, _SBLK)], dbufA, semA).wait()
            pltpu.async_copy(
                dst_hbm.at[pl.ds((j0 + 1) * _SBLK, _SBLK)], dbufB, semB)
            carry = scan(j0, dbufA, carry)

            @pl.when(p < _NB // 2 - 1)
            def _():
                pltpu.async_copy(
                    dst_hbm.at[pl.ds((j0 + 2) * _SBLK, _SBLK)], dbufA, semA)
            pltpu.make_async_copy(
                dst_hbm.at[pl.ds((j0 + 1) * _SBLK, _SBLK)], dbufB, semB).wait()
            carry = scan(j0 + 1, dbufB, carry)
            return carry

        jax.lax.fori_loop(0, _NB // 2, pair, jnp.int32(0))

        @pl.when(nref[0] > 0)
        def _():
            fire()

        @pl.when(lo + _AROWS <= N_TOT)
        def _():
            pltpu.sync_copy(acc, out_hbm.at[pl.ds(lo, _AROWS)])

        @pl.when((lo < N_TOT) & (lo + _AROWS > N_TOT))
        def _():
            pltpu.sync_copy(acc.at[pl.ds(0, N_TOT % _AROWS)],
                            out_hbm.at[pl.ds(lo, N_TOT % _AROWS)])


_sc_params = pltpu.CompilerParams()
if "needs_layout_passes" in pltpu.CompilerParams.__dataclass_fields__:
    _sc_params = _dc.replace(_sc_params, needs_layout_passes=False)

_scatter_agg = pl.kernel(
    _agg_body,
    out_type=jax.ShapeDtypeStruct((N_TOT, _FDIM), jnp.float32),
    mesh=plsc.VectorSubcoreMesh(core_axis_name="c", subcore_axis_name="s"),
    compiler_params=_sc_params,
    scratch_types=[
        pltpu.VMEM((_SBLK,), jnp.int32),
        pltpu.VMEM((_SBLK,), jnp.int32),
        pltpu.VMEM((_LCAP,), jnp.int32),
        pltpu.VMEM((_LCAP + 16,), jnp.int32),
        pltpu.VMEM((_LCAP, _FDIM), jnp.float32),
        pltpu.VMEM((_AROWS, _FDIM), jnp.float32),
        pltpu.SMEM((1,), jnp.int32),
        pltpu.SemaphoreType.DMA,
        pltpu.SemaphoreType.DMA,
        pltpu.SemaphoreType.DMA,
    ],
)


_SROWS = 640
_STAIL = N_TOT - 31 * _SROWS
_L2 = 512
_F2 = _L2 - 16


def _stats_body(epad_hbm, dst_hbm, minit_hbm, zinit_hbm, m_hbm, s_hbm,
                dbufA, dbufB, stagE, stagR, stagS, rows_e, mloc, sloc, nref,
                semA, semB, gsem):
    c = jax.lax.axis_index("c")
    s = jax.lax.axis_index("s")
    wid = c * 16 + s
    lo = wid * _SROWS
    hi = jnp.minimum(lo + _SROWS, N_TOT)
    iota16 = jax.lax.iota(jnp.int32, 16)
    zero16 = jnp.zeros((16,), jnp.int32)

    def clear_stag():
        for q in range(_L2 // 16):
            stagE[pl.ds(16 * q, 16)] = zero16

    def scan_pass(update):
        def fire():
            pltpu.async_copy(epad_hbm.at[stagE], rows_e, gsem).wait()
            nn = nref[0]

            def per_edge(i, carry):
                r = stagR[pl.ds(i, 16)][0]
                sub = stagS[pl.ds(i, 16)][0]
                ivec = jnp.full((16,), i, jnp.int32)
                erow = plsc.load_gather(rows_e, [ivec, sub * 16 + iota16])
                rr = jnp.full((16,), r >> 3, jnp.int32)
                cc = (r & 7) * 16 + iota16
                update(rr, cc, erow)
                return carry

            jax.lax.fori_loop(0, nn, per_edge, jnp.int32(0))
            clear_stag()
            nref[0] = jnp.int32(0)

        def scan(j, buf, carry):
            base = j * _SBLK

            def vec(i, carry2):
                off = i * 16
                d16 = buf[pl.ds(off, 16)]
                inb = (d16 >= lo) & (d16 < hi)
                n0 = nref[0]
                eidx = base + off + iota16
                plsc.store_compressed(stagE.at[pl.ds(n0, 16)],
                                      eidx >> 3, mask=inb)
                plsc.store_compressed(stagS.at[pl.ds(n0, 16)],
                                      eidx & 7, mask=inb)
                plsc.store_compressed(stagR.at[pl.ds(n0, 16)],
                                      d16 - lo, mask=inb)
                nref[0] = n0 + jnp.sum(inb.astype(jnp.int32))

                @pl.when(nref[0] > _F2)
                def _():
                    fire()
                return carry2

            return jax.lax.fori_loop(0, _SBLK // 16, vec, carry)

        clear_stag()
        nref[0] = jnp.int32(0)
        pltpu.async_copy(dst_hbm.at[pl.ds(0, _SBLK)], dbufA, semA)

        def pair(p, carry):
            j0 = 2 * p
            pltpu.make_async_copy(
                dst_hbm.at[pl.ds(j0 * _SBLK, _SBLK)], dbufA, semA).wait()
            pltpu.async_copy(
                dst_hbm.at[pl.ds((j0 + 1) * _SBLK, _SBLK)], dbufB, semB)
            carry = scan(j0, dbufA, carry)

            @pl.when(p < _NB // 2 - 1)
            def _():
                pltpu.async_copy(
                    dst_hbm.at[pl.ds((j0 + 2) * _SBLK, _SBLK)], dbufA, semA)
            pltpu.make_async_copy(
                dst_hbm.at[pl.ds((j0 + 1) * _SBLK, _SBLK)], dbufB, semB).wait()
            carry = scan(j0 + 1, dbufB, carry)
            return carry

        jax.lax.fori_loop(0, _NB // 2, pair, jnp.int32(0))

        @pl.when(nref[0] > 0)
        def _():
            fire()

    pltpu.sync_copy(minit_hbm, mloc)
    pltpu.sync_copy(zinit_hbm, sloc)

    def upd_max(rr, cc, erow):
        old = plsc.load_gather(mloc, [rr, cc])
        plsc.store_scatter(mloc, [rr, cc], jnp.maximum(old, erow))

    scan_pass(upd_max)

    def upd_sum(rr, cc, erow):
        mrow = plsc.load_gather(mloc, [rr, cc])
        old = plsc.load_gather(sloc, [rr, cc])
        plsc.store_scatter(sloc, [rr, cc], old + jnp.exp(erow - mrow))

    scan_pass(upd_sum)

    @pl.when(lo + _SROWS <= N_TOT)
    def _():
        pltpu.sync_copy(mloc, m_hbm.at[pl.ds(wid * (_SROWS // 8), _SROWS // 8)])
        pltpu.sync_copy(sloc, s_hbm.at[pl.ds(wid * (_SROWS // 8), _SROWS // 8)])

    @pl.when(lo + _SROWS > N_TOT)
    def _():
        pltpu.sync_copy(mloc.at[pl.ds(0, 24)],
                        m_hbm.at[pl.ds(wid * (_SROWS // 8), 24)])
        pltpu.sync_copy(sloc.at[pl.ds(0, 24)],
                        s_hbm.at[pl.ds(wid * (_SROWS // 8), 24)])


_edge_stats = pl.kernel(
    _stats_body,
    out_type=(jax.ShapeDtypeStruct((N_TOT // 8 + 4, 128), jnp.float32),
              jax.ShapeDtypeStruct((N_TOT // 8 + 4, 128), jnp.float32)),
    mesh=plsc.VectorSubcoreMesh(core_axis_name="c", subcore_axis_name="s"),
    compiler_params=_sc_params,
    scratch_types=[
        pltpu.VMEM((_SBLK,), jnp.int32),
        pltpu.VMEM((_SBLK,), jnp.int32),
        pltpu.VMEM((_L2,), jnp.int32),
        pltpu.VMEM((_L2 + 16,), jnp.int32),
        pltpu.VMEM((_L2 + 16,), jnp.int32),
        pltpu.VMEM((_L2, 128), jnp.float32),
        pltpu.VMEM((_SROWS // 8, 128), jnp.float32),
        pltpu.VMEM((_SROWS // 8, 128), jnp.float32),
        pltpu.SMEM((1,), jnp.int32),
        pltpu.SemaphoreType.DMA,
        pltpu.SemaphoreType.DMA,
        pltpu.SemaphoreType.DMA,
    ],
)


def _gat(h, src, dst, W, al, ar, zeros, minit, zinit):
    N = h.shape[0]
    ft = (h @ W).reshape(N, NH, HID)
    a1 = jnp.einsum('nhd,hdo->nho', ft, al)
    a2 = jnp.einsum('nhd,hdo->nho', ft, ar)
    e = _leaky(a1[src] + a2[dst])
    emax = jax.ops.segment_max(e, dst, num_segments=N)
    emax = jnp.where(jnp.isfinite(emax), emax, 0.0)
    ee = jnp.exp(e - emax[dst])
    esum = jax.ops.segment_sum(ee, dst, num_segments=N)
    a = ee / (esum[dst] + 1e-9)
    vals = (ft[src] * a).reshape(E, _FDIM)
    out = _scatter_agg(vals, dst, zeros)
    return jax.nn.elu(out)


def _bn_eval(x, g, b):
    return x / jnp.sqrt(1.0 + 1e-5) * g + b


def kernel(x1, x2, edge_index, edge_embeddings, W1, attn_l1, attn_r1, W2,
           attn_l2, attn_r2, bn1_g, bn1_b, lin1_W, lin1_b, bn2_g, bn2_b,
           lin2_W, lin2_b):
    src = edge_index[0].astype(jnp.int32)
    dst = edge_index[1].astype(jnp.int32)
    zeros = jnp.zeros((_AROWS, _FDIM), jnp.float32)
    minit = jnp.full((_SROWS // 8, 128), -3.0e38, jnp.float32)
    zinit = jnp.zeros((_SROWS // 8, 128), jnp.float32)
    x1a = _align(x1, x2.T, x2)
    x2a = _align(x2, x1.T, x1)
    q1 = jnp.concatenate([x1, x1a, x1 - x1a, x1 * x1a], axis=-1)
    q2 = jnp.concatenate([x2, x2a, x2 - x2a, x2 * x2a], axis=-1)
    h = jnp.concatenate([q1, q2], axis=0)
    h = _gat(h, src, dst, W1, attn_l1, attn_r1, zeros, minit, zinit)
    h = _gat(h, src, dst, W2, attn_l2, attn_r2, zeros, minit, zinit)
    hcat = jnp.concatenate([h, edge_embeddings], axis=1)
    gid = jnp.arange(N_TOT) // NPG
    hg = jax.ops.segment_sum(hcat, gid, num_segments=16) / float(NPG)
    xcls = jnp.concatenate([hg[:8], hg[8:]], axis=1)
    y = _bn_eval(xcls, bn1_g, bn1_b)
    y = y @ lin1_W + lin1_b
    y = jax.nn.relu(y)
    y = _bn_eval(y, bn2_g, bn2_b)
    return y @ lin2_W + lin2_b
